# Initial kernel scaffold; baseline (speedup 1.0000x reference)
import jax
import jax.numpy as jnp
from jax.experimental import pallas as pl


def kernel(x, edge_index, Wk, bk, Wm, bm, Wq, bq, W1, b1, gamma, beta, W2, b2):
    raise NotImplementedError("write your pallas kernel here")


# speedup vs baseline: 1.7513x; 1.7513x over previous
import functools
import math

import jax
import jax.numpy as jnp
from jax import lax
from jax.experimental import pallas as pl
from jax.experimental.pallas import tpu as pltpu

N = 10000
E = 320000
D = 128
H = 4
DH = D // H

_HIGH = lax.Precision.HIGHEST


def _proj_body(x_ref, w_ref, b_ref, o_ref):
    o_ref[...] = (
        jnp.dot(x_ref[...], w_ref[...], preferred_element_type=jnp.float32,
                precision=_HIGH)
        + b_ref[...]
    )


def _proj(x, wcat, bcat):
    blk = 2000
    return pl.pallas_call(
        _proj_body,
        grid=(N // blk,),
        in_specs=[
            pl.BlockSpec((blk, D), lambda i: (i, 0)),
            pl.BlockSpec((D, 3 * D), lambda i: (0, 0)),
            pl.BlockSpec((1, 3 * D), lambda i: (0, 0)),
        ],
        out_specs=pl.BlockSpec((blk, 3 * D), lambda i: (i, 0)),
        out_shape=jax.ShapeDtypeStruct((N, 3 * D), jnp.float32),
    )(x, wcat, bcat)


def _mlp_body(a_ref, w1_ref, b1_ref, g_ref, bb_ref, w2_ref, b2_ref, o_ref):
    h = (
        jnp.dot(a_ref[...], w1_ref[...], preferred_element_type=jnp.float32,
                precision=_HIGH)
        + b1_ref[...]
    )
    mean = jnp.mean(h, axis=0, keepdims=True)
    var = jnp.mean(jnp.square(h - mean), axis=0, keepdims=True)
    hn = (h - mean) * jax.lax.rsqrt(var + 1e-5) * g_ref[...] + bb_ref[...]
    hn = jnp.maximum(hn, 0.0)
    o_ref[...] = (
        jnp.dot(hn, w2_ref[...], preferred_element_type=jnp.float32,
                precision=_HIGH)
        + b2_ref[...]
    )


def _mlp(agg, W1, b1, gamma, beta, W2, b2):
    return pl.pallas_call(
        _mlp_body,
        out_shape=jax.ShapeDtypeStruct((N, D), jnp.float32),
    )(agg, W1, b1.reshape(1, D), gamma.reshape(1, D), beta.reshape(1, D),
      W2, b2.reshape(1, D))


def kernel(x, edge_index, Wk, bk, Wm, bm, Wq, bq, W1, b1, gamma, beta, W2, b2):
    src = edge_index[0]
    dst = edge_index[1]
    inv = 1.0 / math.sqrt(DH)
    wcat = jnp.concatenate([Wk, Wm, Wq * inv], axis=1)
    bcat = jnp.concatenate([bk, bm, bq * inv], axis=0).reshape(1, 3 * D)
    kmq = _proj(x, wcat, bcat)
    K = kmq[:, :D]
    M = kmq[:, D:2 * D]
    Q = kmq[:, 2 * D:]

    Qs = jnp.take(Q, src, axis=0)
    Kd = jnp.take(K, dst, axis=0)
    scores = jnp.sum((Qs * Kd).reshape(E, H, DH), axis=2)
    ex = jnp.exp(scores)
    seg_sum = jax.ops.segment_sum(ex, src, num_segments=N)
    deg = jax.ops.segment_sum(jnp.ones((E,), jnp.float32), src, num_segments=N)
    scale = deg[:, None] / (seg_sum + 1e-16)
    Msc = M * jnp.repeat(scale, DH, axis=1)
    Ms = jnp.take(Msc, src, axis=0)
    wrow = Ms * jnp.repeat(ex, DH, axis=1)
    agg = jax.ops.segment_sum(wrow, dst, num_segments=N)

    return _mlp(agg, W1, b1, gamma, beta, W2, b2)


# speedup vs baseline: 5.8616x; 3.3471x over previous
import functools
import math

import jax
import jax.numpy as jnp
from jax import lax
from jax.experimental import pallas as pl
from jax.experimental.pallas import tpu as pltpu
from jax.experimental.pallas import tpu_sc as plsc

N = 10000
E = 320000
D = 128
H = 4
DH = D // H

_NC = 2
_NS = 16
_NW = _NC * _NS
_GW = 128
_BLK = 512
_GPB = _BLK // _GW
_NBLK = E // _BLK
_NPAD = 10240
_NROWS_PER_SUB = _NPAD // _NS

_HIGH = lax.Precision.HIGHEST
_f32 = jnp.float32


def _mesh():
    return plsc.VectorSubcoreMesh(core_axis_name="c", subcore_axis_name="s")


def _proj_body(x_ref, w_ref, b_ref, k_ref, m_ref, q_ref):
    kmq = (
        jnp.dot(x_ref[...], w_ref[...], preferred_element_type=_f32,
                precision=_HIGH)
        + b_ref[...]
    )
    k_ref[...] = kmq[:, :D]
    m_ref[...] = kmq[:, D:2 * D]
    q_ref[...] = kmq[:, 2 * D:]


def _proj(x, wcat, bcat):
    blk = 2000
    sds = jax.ShapeDtypeStruct((N, D), _f32)
    return pl.pallas_call(
        _proj_body,
        grid=(N // blk,),
        in_specs=[
            pl.BlockSpec((blk, D), lambda i: (i, 0)),
            pl.BlockSpec((D, 3 * D), lambda i: (0, 0)),
            pl.BlockSpec((1, 3 * D), lambda i: (0, 0)),
        ],
        out_specs=[pl.BlockSpec((blk, D), lambda i: (i, 0))] * 3,
        out_shape=[sds, sds, sds],
    )(x, wcat, bcat)


def _sc_gather2(t1, t2, i1_1d, i2_1d):
    sds = jax.ShapeDtypeStruct((E, D), _f32)

    @functools.partial(
        pl.kernel,
        out_type=[sds, sds],
        mesh=_mesh(),
        scratch_types=[
            pltpu.VMEM((_BLK,), jnp.int32),
            pltpu.VMEM((_BLK, D), _f32),
            pltpu.SemaphoreType.DMA,
        ],
    )
    def k(t1_hbm, t2_hbm, i1_hbm, i2_hbm, o1_hbm, o2_hbm, idx_v, rows_v, sem):
        wid = lax.axis_index("s") * _NC + lax.axis_index("c")

        @pl.loop(wid, _NBLK, step=_NW)
        def _(b):
            for t_hbm, i_hbm, o_hbm in ((t1_hbm, i1_hbm, o1_hbm),
                                        (t2_hbm, i2_hbm, o2_hbm)):
                pltpu.sync_copy(i_hbm.at[pl.ds(b * _BLK, _BLK)], idx_v)
                cps = [
                    pltpu.async_copy(t_hbm.at[idx_v.at[pl.ds(j * _GW, _GW)]],
                                     rows_v.at[pl.ds(j * _GW, _GW)], sem)
                    for j in range(_GPB)
                ]
                for c in cps:
                    c.wait()
                pltpu.sync_copy(rows_v, o_hbm.at[pl.ds(b * _BLK, _BLK)])

    return k(t1, t2, i1_1d, i2_1d)


def _sc_gather1(t1, i1_1d):
    sds = jax.ShapeDtypeStruct((E, D), _f32)

    @functools.partial(
        pl.kernel,
        out_type=sds,
        mesh=_mesh(),
        scratch_types=[
            pltpu.VMEM((_BLK,), jnp.int32),
            pltpu.VMEM((_BLK, D), _f32),
            pltpu.SemaphoreType.DMA,
        ],
    )
    def k(t1_hbm, i1_hbm, o1_hbm, idx_v, rows_v, sem):
        wid = lax.axis_index("s") * _NC + lax.axis_index("c")

        @pl.loop(wid, _NBLK, step=_NW)
        def _(b):
            pltpu.sync_copy(i1_hbm.at[pl.ds(b * _BLK, _BLK)], idx_v)
            cps = [
                pltpu.async_copy(t1_hbm.at[idx_v.at[pl.ds(j * _GW, _GW)]],
                                 rows_v.at[pl.ds(j * _GW, _GW)], sem)
                for j in range(_GPB)
            ]
            for c in cps:
                c.wait()
            pltpu.sync_copy(rows_v, o1_hbm.at[pl.ds(b * _BLK, _BLK)])

    return k(t1, i1_1d)


_SBLK = 128
_SNBLK = E // _SBLK

def _sc_scatter_add(rows, idx_1d, zeros, width):
    @functools.partial(
        pl.kernel,
        out_type=jax.ShapeDtypeStruct((_NC, _NPAD, width), _f32),
        mesh=_mesh(),
        scratch_types=[
            pltpu.VMEM((_SBLK,), jnp.int32),
            pltpu.VMEM((_SBLK, width), _f32),
            pltpu.VMEM_SHARED((_NPAD, width), _f32),
            pltpu.SemaphoreType.DMA,
        ],
    )
    def k(rows_hbm, idx_hbm, z_hbm, out_hbm, idx_v, rows_v, acc_sh, sem):
        cid = lax.axis_index("c")
        sid = lax.axis_index("s")
        r0 = sid * _NROWS_PER_SUB
        pltpu.sync_copy(z_hbm.at[pl.ds(r0, _NROWS_PER_SUB)],
                        acc_sh.at[pl.ds(r0, _NROWS_PER_SUB)])
        plsc.subcore_barrier()
        wid = sid * _NC + cid

        @pl.loop(wid, _SNBLK, step=_NW)
        def _(b):
            pltpu.sync_copy(idx_hbm.at[pl.ds(b * _SBLK, _SBLK)], idx_v)
            pltpu.sync_copy(rows_hbm.at[pl.ds(b * _SBLK, _SBLK)], rows_v)
            pltpu.sync_copy(rows_v, acc_sh.at[idx_v], add=True)

        plsc.subcore_barrier()
        pltpu.sync_copy(acc_sh.at[pl.ds(r0, _NROWS_PER_SUB)],
                        out_hbm.at[cid, pl.ds(r0, _NROWS_PER_SUB)])

    return k(rows, idx_1d, zeros)


def _scores_body(qs_ref, kd_ref, s8_ref, o_ref):
    p = qs_ref[...] * kd_ref[...]
    dots = jnp.dot(p, s8_ref[...], preferred_element_type=_f32,
                   precision=_HIGH)
    col = lax.broadcasted_iota(jnp.int32, dots.shape, 1)
    o_ref[...] = jnp.where(col < H, jnp.exp(dots),
                           jnp.where(col == H, 1.0, 0.0))


def _scores(qs, kd, s8):
    blk = 4000
    return pl.pallas_call(
        _scores_body,
        grid=(E // blk,),
        in_specs=[
            pl.BlockSpec((blk, D), lambda i: (i, 0)),
            pl.BlockSpec((blk, D), lambda i: (i, 0)),
            pl.BlockSpec((D, D), lambda i: (0, 0)),
        ],
        out_specs=pl.BlockSpec((blk, D), lambda i: (i, 0)),
        out_shape=jax.ShapeDtypeStruct((E, D), _f32),
    )(qs, kd, s8)


def _msc_body(ss_ref, m_ref, r16_ref, o_ref):
    ss = (ss_ref[0] + ss_ref[1])[:, :16]
    col = lax.broadcasted_iota(jnp.int32, ss.shape, 1)
    deg = lax.broadcast_in_dim(ss[:, H], ss.shape, (0,))
    scale16 = jnp.where(col < H, deg / (ss + 1e-16), 0.0)
    scale128 = jnp.dot(scale16, r16_ref[...], preferred_element_type=_f32,
                       precision=_HIGH)
    o_ref[...] = m_ref[...] * scale128


def _msc(ssp, m, r16):
    blk = 2000
    return pl.pallas_call(
        _msc_body,
        grid=(N // blk,),
        in_specs=[
            pl.BlockSpec((_NC, blk, D), lambda i: (0, i, 0)),
            pl.BlockSpec((blk, D), lambda i: (i, 0)),
            pl.BlockSpec((16, D), lambda i: (0, 0)),
        ],
        out_specs=pl.BlockSpec((blk, D), lambda i: (i, 0)),
        out_shape=jax.ShapeDtypeStruct((N, D), _f32),
    )(ssp, m, r16)


def _wrow_body(ms_ref, exd_ref, r16_ref, o_ref):
    ex128 = jnp.dot(exd_ref[:, :16], r16_ref[...], preferred_element_type=_f32,
                    precision=_HIGH)
    o_ref[...] = ms_ref[...] * ex128


def _wrow(ms, exd, r16):
    blk = 4000
    return pl.pallas_call(
        _wrow_body,
        grid=(E // blk,),
        in_specs=[
            pl.BlockSpec((blk, D), lambda i: (i, 0)),
            pl.BlockSpec((blk, D), lambda i: (i, 0)),
            pl.BlockSpec((16, D), lambda i: (0, 0)),
        ],
        out_specs=pl.BlockSpec((blk, D), lambda i: (i, 0)),
        out_shape=jax.ShapeDtypeStruct((E, D), _f32),
    )(ms, exd, r16)


def _mlp_body(a_ref, w1_ref, b1_ref, g_ref, bb_ref, w2_ref, b2_ref, o_ref):
    a = a_ref[0] + a_ref[1]
    h = (
        jnp.dot(a, w1_ref[...], preferred_element_type=_f32, precision=_HIGH)
        + b1_ref[...]
    )
    mean = jnp.mean(h, axis=0, keepdims=True)
    var = jnp.mean(jnp.square(h - mean), axis=0, keepdims=True)
    hn = (h - mean) * jax.lax.rsqrt(var + 1e-5) * g_ref[...] + bb_ref[...]
    hn = jnp.maximum(hn, 0.0)
    o_ref[...] = (
        jnp.dot(hn, w2_ref[...], preferred_element_type=_f32, precision=_HIGH)
        + b2_ref[...]
    )


def _mlp(aggp, W1, b1, gamma, beta, W2, b2):
    return pl.pallas_call(
        _mlp_body,
        out_shape=jax.ShapeDtypeStruct((N, D), _f32),
    )(aggp, W1, b1.reshape(1, D), gamma.reshape(1, D), beta.reshape(1, D),
      W2, b2.reshape(1, D))


def kernel(x, edge_index, Wk, bk, Wm, bm, Wq, bq, W1, b1, gamma, beta, W2, b2):
    src1d = edge_index[0]
    dst1d = edge_index[1]
    inv = 1.0 / math.sqrt(DH)
    wcat = jnp.concatenate([Wk, Wm, Wq * inv], axis=1)
    bcat = jnp.concatenate([bk, bm, bq * inv], axis=0).reshape(1, 3 * D)
    K, M, Q = _proj(x, wcat, bcat)

    d_iota = jnp.arange(D, dtype=jnp.int32)
    s8 = (d_iota[:, None] // DH == jnp.arange(D)[None, :]).astype(_f32)
    r16 = (jnp.arange(16, dtype=jnp.int32)[:, None] == d_iota[None, :] // DH
           ).astype(_f32)

    qs, kd = _sc_gather2(Q, K, src1d, dst1d)
    exd = _scores(qs, kd, s8)
    z128 = jnp.zeros((_NPAD, D), _f32)
    ssp = _sc_scatter_add(exd, src1d, z128, D)
    msc = _msc(ssp[:, :N], M, r16)
    ms = _sc_gather1(msc, src1d)
    wr = _wrow(ms, exd, r16)
    aggp = _sc_scatter_add(wr, dst1d, z128, D)
    return _mlp(aggp[:, :N], W1, b1, gamma, beta, W2, b2)


# speedup vs baseline: 6.2306x; 1.0630x over previous
import functools
import math

import jax
import jax.numpy as jnp
from jax import lax
from jax.experimental import pallas as pl
from jax.experimental.pallas import tpu as pltpu
from jax.experimental.pallas import tpu_sc as plsc

N = 10000
E = 320000
D = 128
H = 4
DH = D // H

_NC = 2
_NS = 16
_NW = _NC * _NS
_GW = 128
_BLK = 512
_GPB = _BLK // _GW
_NBLK = E // _BLK
_NPAD = 10240
_NROWS_PER_SUB = _NPAD // _NS

_HIGH = lax.Precision.HIGHEST
_f32 = jnp.float32


def _mesh():
    return plsc.VectorSubcoreMesh(core_axis_name="c", subcore_axis_name="s")


def _proj_body(x_ref, w_ref, b_ref, k_ref, m_ref, q_ref):
    kmq = (
        jnp.dot(x_ref[...], w_ref[...], preferred_element_type=_f32,
                precision=_HIGH)
        + b_ref[...]
    )
    k_ref[...] = kmq[:, :D]
    m_ref[...] = kmq[:, D:2 * D]
    q_ref[...] = kmq[:, 2 * D:]


def _proj(x, wcat, bcat):
    blk = 2000
    sds = jax.ShapeDtypeStruct((N, D), _f32)
    return pl.pallas_call(
        _proj_body,
        grid=(N // blk,),
        in_specs=[
            pl.BlockSpec((blk, D), lambda i: (i, 0)),
            pl.BlockSpec((D, 3 * D), lambda i: (0, 0)),
            pl.BlockSpec((1, 3 * D), lambda i: (0, 0)),
        ],
        out_specs=[pl.BlockSpec((blk, D), lambda i: (i, 0))] * 3,
        out_shape=[sds, sds, sds],
    )(x, wcat, bcat)


_G2BLK = 256
_G2PB = _G2BLK // _GW
_G2NBLK = E // _G2BLK

def _sc_gather2(t1, t2, i1_1d, i2_1d):
    sds = jax.ShapeDtypeStruct((E, D), _f32)

    @functools.partial(
        pl.kernel,
        out_type=[sds, sds],
        mesh=_mesh(),
        scratch_types=[
            pltpu.VMEM((_G2BLK,), jnp.int32),
            pltpu.VMEM((_G2BLK,), jnp.int32),
            pltpu.VMEM((_G2BLK, D), _f32),
            pltpu.VMEM((_G2BLK, D), _f32),
            pltpu.SemaphoreType.DMA,
            pltpu.SemaphoreType.DMA,
        ],
    )
    def k(t1_hbm, t2_hbm, i1_hbm, i2_hbm, o1_hbm, o2_hbm,
          ia_v, ib_v, ra_v, rb_v, sema, semb):
        wid = lax.axis_index("s") * _NC + lax.axis_index("c")

        @pl.loop(wid, _G2NBLK, step=_NW)
        def _(b):
            pltpu.sync_copy(i1_hbm.at[pl.ds(b * _G2BLK, _G2BLK)], ia_v)
            ca = [
                pltpu.async_copy(t1_hbm.at[ia_v.at[pl.ds(j * _GW, _GW)]],
                                 ra_v.at[pl.ds(j * _GW, _GW)], sema)
                for j in range(_G2PB)
            ]
            pltpu.sync_copy(i2_hbm.at[pl.ds(b * _G2BLK, _G2BLK)], ib_v)
            cb = [
                pltpu.async_copy(t2_hbm.at[ib_v.at[pl.ds(j * _GW, _GW)]],
                                 rb_v.at[pl.ds(j * _GW, _GW)], semb)
                for j in range(_G2PB)
            ]
            for c in ca:
                c.wait()
            pltpu.sync_copy(ra_v, o1_hbm.at[pl.ds(b * _G2BLK, _G2BLK)])
            for c in cb:
                c.wait()
            pltpu.sync_copy(rb_v, o2_hbm.at[pl.ds(b * _G2BLK, _G2BLK)])

    return k(t1, t2, i1_1d, i2_1d)


def _sc_gather1(t1, i1_1d):
    sds = jax.ShapeDtypeStruct((E, D), _f32)

    @functools.partial(
        pl.kernel,
        out_type=sds,
        mesh=_mesh(),
        scratch_types=[
            pltpu.VMEM((_G2BLK,), jnp.int32),
            pltpu.VMEM((_G2BLK,), jnp.int32),
            pltpu.VMEM((_G2BLK, D), _f32),
            pltpu.VMEM((_G2BLK, D), _f32),
            pltpu.SemaphoreType.DMA,
            pltpu.SemaphoreType.DMA,
        ],
    )
    def k(t1_hbm, i1_hbm, o1_hbm, ia_v, ib_v, ra_v, rb_v, sema, semb):
        wid = lax.axis_index("s") * _NC + lax.axis_index("c")

        @pl.loop(wid, _G2NBLK, step=2 * _NW)
        def _(b):
            b1 = b + _NW
            pltpu.sync_copy(i1_hbm.at[pl.ds(b * _G2BLK, _G2BLK)], ia_v)
            ca = [
                pltpu.async_copy(t1_hbm.at[ia_v.at[pl.ds(j * _GW, _GW)]],
                                 ra_v.at[pl.ds(j * _GW, _GW)], sema)
                for j in range(_G2PB)
            ]

            @pl.when(b1 < _G2NBLK)
            def _():
                pltpu.sync_copy(i1_hbm.at[pl.ds(b1 * _G2BLK, _G2BLK)], ib_v)
                for j in range(_G2PB):
                    pltpu.async_copy(t1_hbm.at[ib_v.at[pl.ds(j * _GW, _GW)]],
                                     rb_v.at[pl.ds(j * _GW, _GW)], semb)

            for c in ca:
                c.wait()
            pltpu.sync_copy(ra_v, o1_hbm.at[pl.ds(b * _G2BLK, _G2BLK)])

            @pl.when(b1 < _G2NBLK)
            def _():
                for j in range(_G2PB):
                    pltpu.make_async_copy(
                        t1_hbm.at[ib_v.at[pl.ds(j * _GW, _GW)]],
                        rb_v.at[pl.ds(j * _GW, _GW)], semb).wait()
                pltpu.sync_copy(rb_v, o1_hbm.at[pl.ds(b1 * _G2BLK, _G2BLK)])

    return k(t1, i1_1d)


_SBLK = 128
_SNBLK = E // _SBLK

def _sc_scatter_add(rows, idx_1d, zeros, width):
    @functools.partial(
        pl.kernel,
        out_type=jax.ShapeDtypeStruct((_NC, _NPAD, width), _f32),
        mesh=_mesh(),
        scratch_types=[
            pltpu.VMEM((_SBLK,), jnp.int32),
            pltpu.VMEM((_SBLK,), jnp.int32),
            pltpu.VMEM((_SBLK, width), _f32),
            pltpu.VMEM((_SBLK, width), _f32),
            pltpu.VMEM_SHARED((_NPAD, width), _f32),
            pltpu.SemaphoreType.DMA,
            pltpu.SemaphoreType.DMA,
        ],
    )
    def k(rows_hbm, idx_hbm, z_hbm, out_hbm, ia_v, ib_v, ra_v, rb_v,
          acc_sh, sema, semb):
        cid = lax.axis_index("c")
        sid = lax.axis_index("s")
        r0 = sid * _NROWS_PER_SUB
        pltpu.sync_copy(z_hbm.at[pl.ds(r0, _NROWS_PER_SUB)],
                        acc_sh.at[pl.ds(r0, _NROWS_PER_SUB)])
        plsc.subcore_barrier()
        wid = sid * _NC + cid

        @pl.loop(wid, _SNBLK, step=2 * _NW)
        def _(b):
            b1 = b + _NW
            pltpu.sync_copy(idx_hbm.at[pl.ds(b * _SBLK, _SBLK)], ia_v)
            pltpu.sync_copy(rows_hbm.at[pl.ds(b * _SBLK, _SBLK)], ra_v)
            sca = pltpu.async_copy(ra_v, acc_sh.at[ia_v], sema, add=True)

            @pl.when(b1 < _SNBLK)
            def _():
                pltpu.sync_copy(idx_hbm.at[pl.ds(b1 * _SBLK, _SBLK)], ib_v)
                pltpu.sync_copy(rows_hbm.at[pl.ds(b1 * _SBLK, _SBLK)], rb_v)
                pltpu.async_copy(rb_v, acc_sh.at[ib_v], semb, add=True)

            sca.wait()

            @pl.when(b1 < _SNBLK)
            def _():
                pltpu.make_async_copy(rb_v, acc_sh.at[ib_v], semb).wait()

        plsc.subcore_barrier()
        pltpu.sync_copy(acc_sh.at[pl.ds(r0, _NROWS_PER_SUB)],
                        out_hbm.at[cid, pl.ds(r0, _NROWS_PER_SUB)])

    return k(rows, idx_1d, zeros)


def _scores_body(qs_ref, kd_ref, s8_ref, o_ref):
    p = qs_ref[...] * kd_ref[...]
    dots = jnp.dot(p, s8_ref[...], preferred_element_type=_f32,
                   precision=_HIGH)
    col = lax.broadcasted_iota(jnp.int32, dots.shape, 1)
    o_ref[...] = jnp.where(col < H, jnp.exp(dots),
                           jnp.where(col == H, 1.0, 0.0))


def _scores(qs, kd, s8):
    blk = 4000
    return pl.pallas_call(
        _scores_body,
        grid=(E // blk,),
        in_specs=[
            pl.BlockSpec((blk, D), lambda i: (i, 0)),
            pl.BlockSpec((blk, D), lambda i: (i, 0)),
            pl.BlockSpec((D, D), lambda i: (0, 0)),
        ],
        out_specs=pl.BlockSpec((blk, D), lambda i: (i, 0)),
        out_shape=jax.ShapeDtypeStruct((E, D), _f32),
    )(qs, kd, s8)


def _msc_body(ss_ref, m_ref, r16_ref, o_ref):
    ss = (ss_ref[0] + ss_ref[1])[:, :16]
    col = lax.broadcasted_iota(jnp.int32, ss.shape, 1)
    deg = lax.broadcast_in_dim(ss[:, H], ss.shape, (0,))
    scale16 = jnp.where(col < H, deg / (ss + 1e-16), 0.0)
    scale128 = jnp.dot(scale16, r16_ref[...], preferred_element_type=_f32,
                       precision=_HIGH)
    o_ref[...] = m_ref[...] * scale128


def _msc(ssp, m, r16):
    blk = 2000
    return pl.pallas_call(
        _msc_body,
        grid=(N // blk,),
        in_specs=[
            pl.BlockSpec((_NC, blk, D), lambda i: (0, i, 0)),
            pl.BlockSpec((blk, D), lambda i: (i, 0)),
            pl.BlockSpec((16, D), lambda i: (0, 0)),
        ],
        out_specs=pl.BlockSpec((blk, D), lambda i: (i, 0)),
        out_shape=jax.ShapeDtypeStruct((N, D), _f32),
    )(ssp, m, r16)


def _wrow_body(ms_ref, exd_ref, r16_ref, o_ref):
    ex128 = jnp.dot(exd_ref[:, :16], r16_ref[...], preferred_element_type=_f32,
                    precision=_HIGH)
    o_ref[...] = ms_ref[...] * ex128


def _wrow(ms, exd, r16):
    blk = 4000
    return pl.pallas_call(
        _wrow_body,
        grid=(E // blk,),
        in_specs=[
            pl.BlockSpec((blk, D), lambda i: (i, 0)),
            pl.BlockSpec((blk, D), lambda i: (i, 0)),
            pl.BlockSpec((16, D), lambda i: (0, 0)),
        ],
        out_specs=pl.BlockSpec((blk, D), lambda i: (i, 0)),
        out_shape=jax.ShapeDtypeStruct((E, D), _f32),
    )(ms, exd, r16)


def _mlp_body(a_ref, w1_ref, b1_ref, g_ref, bb_ref, w2_ref, b2_ref, o_ref):
    a = a_ref[0] + a_ref[1]
    h = (
        jnp.dot(a, w1_ref[...], preferred_element_type=_f32, precision=_HIGH)
        + b1_ref[...]
    )
    mean = jnp.mean(h, axis=0, keepdims=True)
    var = jnp.mean(jnp.square(h - mean), axis=0, keepdims=True)
    hn = (h - mean) * jax.lax.rsqrt(var + 1e-5) * g_ref[...] + bb_ref[...]
    hn = jnp.maximum(hn, 0.0)
    o_ref[...] = (
        jnp.dot(hn, w2_ref[...], preferred_element_type=_f32, precision=_HIGH)
        + b2_ref[...]
    )


def _mlp(aggp, W1, b1, gamma, beta, W2, b2):
    return pl.pallas_call(
        _mlp_body,
        out_shape=jax.ShapeDtypeStruct((N, D), _f32),
    )(aggp, W1, b1.reshape(1, D), gamma.reshape(1, D), beta.reshape(1, D),
      W2, b2.reshape(1, D))


def kernel(x, edge_index, Wk, bk, Wm, bm, Wq, bq, W1, b1, gamma, beta, W2, b2):
    src1d = edge_index[0]
    dst1d = edge_index[1]
    inv = 1.0 / math.sqrt(DH)
    wcat = jnp.concatenate([Wk, Wm, Wq * inv], axis=1)
    bcat = jnp.concatenate([bk, bm, bq * inv], axis=0).reshape(1, 3 * D)
    K, M, Q = _proj(x, wcat, bcat)

    d_iota = jnp.arange(D, dtype=jnp.int32)
    s8 = (d_iota[:, None] // DH == jnp.arange(D)[None, :]).astype(_f32)
    r16 = (jnp.arange(16, dtype=jnp.int32)[:, None] == d_iota[None, :] // DH
           ).astype(_f32)

    qs, kd = _sc_gather2(Q, K, src1d, dst1d)
    exd = _scores(qs, kd, s8)
    z128 = jnp.zeros((_NPAD, D), _f32)
    ssp = _sc_scatter_add(exd, src1d, z128, D)
    msc = _msc(ssp[:, :N], M, r16)
    ms = _sc_gather1(msc, src1d)
    wr = _wrow(ms, exd, r16)
    aggp = _sc_scatter_add(wr, dst1d, z128, D)
    return _mlp(aggp[:, :N], W1, b1, gamma, beta, W2, b2)


# speedup vs baseline: 6.9315x; 1.1125x over previous
import functools
import math

import jax
import jax.numpy as jnp
from jax import lax
from jax.experimental import pallas as pl
from jax.experimental.pallas import tpu as pltpu
from jax.experimental.pallas import tpu_sc as plsc

N = 10000
E = 320000
D = 128
H = 4
DH = D // H

_NC = 2
_NS = 16
_NW = _NC * _NS
_GW = 128
_BLK = 512
_GPB = _BLK // _GW
_NBLK = E // _BLK
_NPAD = 10240
_NROWS_PER_SUB = _NPAD // _NS

_HIGH = lax.Precision.HIGHEST
_f32 = jnp.float32


def _mesh():
    return plsc.VectorSubcoreMesh(core_axis_name="c", subcore_axis_name="s")


def _proj_body(x_ref, w_ref, b_ref, k_ref, m_ref, q_ref):
    kmq = (
        jnp.dot(x_ref[...], w_ref[...], preferred_element_type=_f32,
                precision=_HIGH)
        + b_ref[...]
    )
    k_ref[...] = kmq[:, :D]
    m_ref[...] = kmq[:, D:2 * D]
    q_ref[...] = kmq[:, 2 * D:]


def _proj(x, wcat, bcat):
    blk = 2000
    sds = jax.ShapeDtypeStruct((N, D), _f32)
    return pl.pallas_call(
        _proj_body,
        grid=(N // blk,),
        in_specs=[
            pl.BlockSpec((blk, D), lambda i: (i, 0)),
            pl.BlockSpec((D, 3 * D), lambda i: (0, 0)),
            pl.BlockSpec((1, 3 * D), lambda i: (0, 0)),
        ],
        out_specs=[pl.BlockSpec((blk, D), lambda i: (i, 0))] * 3,
        out_shape=[sds, sds, sds],
    )(x, wcat, bcat)


_G2BLK = 256
_G2PB = _G2BLK // _GW
_G2NBLK = E // _G2BLK

def _sc_gather2(t1, t2, i1_1d, i2_1d, ne=E):
    sds = jax.ShapeDtypeStruct((ne, D), _f32)
    nblk = ne // _G2BLK

    @functools.partial(
        pl.kernel,
        out_type=[sds, sds],
        mesh=_mesh(),
        scratch_types=[
            pltpu.VMEM((_G2BLK,), jnp.int32),
            pltpu.VMEM((_G2BLK,), jnp.int32),
            pltpu.VMEM((_G2BLK, D), _f32),
            pltpu.VMEM((_G2BLK, D), _f32),
            pltpu.SemaphoreType.DMA,
            pltpu.SemaphoreType.DMA,
        ],
    )
    def k(t1_hbm, t2_hbm, i1_hbm, i2_hbm, o1_hbm, o2_hbm,
          ia_v, ib_v, ra_v, rb_v, sema, semb):
        wid = lax.axis_index("s") * _NC + lax.axis_index("c")

        @pl.loop(wid, nblk, step=_NW)
        def _(b):
            pltpu.sync_copy(i1_hbm.at[pl.ds(b * _G2BLK, _G2BLK)], ia_v)
            ca = [
                pltpu.async_copy(t1_hbm.at[ia_v.at[pl.ds(j * _GW, _GW)]],
                                 ra_v.at[pl.ds(j * _GW, _GW)], sema)
                for j in range(_G2PB)
            ]
            pltpu.sync_copy(i2_hbm.at[pl.ds(b * _G2BLK, _G2BLK)], ib_v)
            cb = [
                pltpu.async_copy(t2_hbm.at[ib_v.at[pl.ds(j * _GW, _GW)]],
                                 rb_v.at[pl.ds(j * _GW, _GW)], semb)
                for j in range(_G2PB)
            ]
            for c in ca:
                c.wait()
            pltpu.sync_copy(ra_v, o1_hbm.at[pl.ds(b * _G2BLK, _G2BLK)])
            for c in cb:
                c.wait()
            pltpu.sync_copy(rb_v, o2_hbm.at[pl.ds(b * _G2BLK, _G2BLK)])

    return k(t1, t2, i1_1d, i2_1d)


def _sc_gather1(t1, i1_1d, ne=E):
    sds = jax.ShapeDtypeStruct((ne, D), _f32)
    nblk = ne // _G2BLK

    @functools.partial(
        pl.kernel,
        out_type=sds,
        mesh=_mesh(),
        scratch_types=[
            pltpu.VMEM((_G2BLK,), jnp.int32),
            pltpu.VMEM((_G2BLK,), jnp.int32),
            pltpu.VMEM((_G2BLK, D), _f32),
            pltpu.VMEM((_G2BLK, D), _f32),
            pltpu.SemaphoreType.DMA,
            pltpu.SemaphoreType.DMA,
        ],
    )
    def k(t1_hbm, i1_hbm, o1_hbm, ia_v, ib_v, ra_v, rb_v, sema, semb):
        wid = lax.axis_index("s") * _NC + lax.axis_index("c")

        @pl.loop(wid, nblk, step=2 * _NW)
        def _(b):
            b1 = b + _NW
            pltpu.sync_copy(i1_hbm.at[pl.ds(b * _G2BLK, _G2BLK)], ia_v)
            ca = [
                pltpu.async_copy(t1_hbm.at[ia_v.at[pl.ds(j * _GW, _GW)]],
                                 ra_v.at[pl.ds(j * _GW, _GW)], sema)
                for j in range(_G2PB)
            ]

            @pl.when(b1 < nblk)
            def _():
                pltpu.sync_copy(i1_hbm.at[pl.ds(b1 * _G2BLK, _G2BLK)], ib_v)
                for j in range(_G2PB):
                    pltpu.async_copy(t1_hbm.at[ib_v.at[pl.ds(j * _GW, _GW)]],
                                     rb_v.at[pl.ds(j * _GW, _GW)], semb)

            for c in ca:
                c.wait()
            pltpu.sync_copy(ra_v, o1_hbm.at[pl.ds(b * _G2BLK, _G2BLK)])

            @pl.when(b1 < nblk)
            def _():
                for j in range(_G2PB):
                    pltpu.make_async_copy(
                        t1_hbm.at[ib_v.at[pl.ds(j * _GW, _GW)]],
                        rb_v.at[pl.ds(j * _GW, _GW)], semb).wait()
                pltpu.sync_copy(rb_v, o1_hbm.at[pl.ds(b1 * _G2BLK, _G2BLK)])

    return k(t1, i1_1d)


_SBLK = 128
_SNBLK = E // _SBLK

def _sc_scatter_add(rows, idx_1d, zeros, width, ne=E):
    nblk = ne // _SBLK

    @functools.partial(
        pl.kernel,
        out_type=jax.ShapeDtypeStruct((_NC, _NPAD, width), _f32),
        mesh=_mesh(),
        scratch_types=[
            pltpu.VMEM((_SBLK,), jnp.int32),
            pltpu.VMEM((_SBLK,), jnp.int32),
            pltpu.VMEM((_SBLK, width), _f32),
            pltpu.VMEM((_SBLK, width), _f32),
            pltpu.VMEM_SHARED((_NPAD, width), _f32),
            pltpu.SemaphoreType.DMA,
            pltpu.SemaphoreType.DMA,
        ],
    )
    def k(rows_hbm, idx_hbm, z_hbm, out_hbm, ia_v, ib_v, ra_v, rb_v,
          acc_sh, sema, semb):
        cid = lax.axis_index("c")
        sid = lax.axis_index("s")
        r0 = sid * _NROWS_PER_SUB
        pltpu.sync_copy(z_hbm.at[pl.ds(r0, _NROWS_PER_SUB)],
                        acc_sh.at[pl.ds(r0, _NROWS_PER_SUB)])
        plsc.subcore_barrier()
        wid = sid * _NC + cid

        @pl.loop(wid, nblk, step=2 * _NW)
        def _(b):
            b1 = b + _NW
            pltpu.sync_copy(idx_hbm.at[pl.ds(b * _SBLK, _SBLK)], ia_v)
            pltpu.sync_copy(rows_hbm.at[pl.ds(b * _SBLK, _SBLK)], ra_v)
            sca = pltpu.async_copy(ra_v, acc_sh.at[ia_v], sema, add=True)

            @pl.when(b1 < nblk)
            def _():
                pltpu.sync_copy(idx_hbm.at[pl.ds(b1 * _SBLK, _SBLK)], ib_v)
                pltpu.sync_copy(rows_hbm.at[pl.ds(b1 * _SBLK, _SBLK)], rb_v)
                pltpu.async_copy(rb_v, acc_sh.at[ib_v], semb, add=True)

            sca.wait()

            @pl.when(b1 < nblk)
            def _():
                pltpu.make_async_copy(rb_v, acc_sh.at[ib_v], semb).wait()

        plsc.subcore_barrier()
        pltpu.sync_copy(acc_sh.at[pl.ds(r0, _NROWS_PER_SUB)],
                        out_hbm.at[cid, pl.ds(r0, _NROWS_PER_SUB)])

    return k(rows, idx_1d, zeros)


def _scores_body(qs_ref, kd_ref, s8_ref, o_ref):
    p = qs_ref[...] * kd_ref[...]
    dots = jnp.dot(p, s8_ref[...], preferred_element_type=_f32,
                   precision=_HIGH)
    col = lax.broadcasted_iota(jnp.int32, dots.shape, 1)
    o_ref[...] = jnp.where(col < H, jnp.exp(dots),
                           jnp.where(col == H, 1.0, 0.0))


def _scores(qs, kd, s8):
    blk = 4000
    ne = qs.shape[0]
    return pl.pallas_call(
        _scores_body,
        grid=(ne // blk,),
        in_specs=[
            pl.BlockSpec((blk, D), lambda i: (i, 0)),
            pl.BlockSpec((blk, D), lambda i: (i, 0)),
            pl.BlockSpec((D, D), lambda i: (0, 0)),
        ],
        out_specs=pl.BlockSpec((blk, D), lambda i: (i, 0)),
        out_shape=jax.ShapeDtypeStruct((ne, D), _f32),
    )(qs, kd, s8)


def _msc_body(ssa_ref, ssb_ref, m_ref, r16_ref, o_ref):
    ss = (ssa_ref[0] + ssa_ref[1] + ssb_ref[0] + ssb_ref[1])[:, :16]
    col = lax.broadcasted_iota(jnp.int32, ss.shape, 1)
    deg = lax.broadcast_in_dim(ss[:, H], ss.shape, (0,))
    scale16 = jnp.where(col < H, deg / (ss + 1e-16), 0.0)
    scale128 = jnp.dot(scale16, r16_ref[...], preferred_element_type=_f32,
                       precision=_HIGH)
    o_ref[...] = m_ref[...] * scale128


def _msc(sspa, sspb, m, r16):
    blk = 2000
    return pl.pallas_call(
        _msc_body,
        grid=(N // blk,),
        in_specs=[
            pl.BlockSpec((_NC, blk, D), lambda i: (0, i, 0)),
            pl.BlockSpec((_NC, blk, D), lambda i: (0, i, 0)),
            pl.BlockSpec((blk, D), lambda i: (i, 0)),
            pl.BlockSpec((16, D), lambda i: (0, 0)),
        ],
        out_specs=pl.BlockSpec((blk, D), lambda i: (i, 0)),
        out_shape=jax.ShapeDtypeStruct((N, D), _f32),
    )(sspa, sspb, m, r16)


def _wrow_body(ms_ref, exd_ref, r16_ref, o_ref):
    ex128 = jnp.dot(exd_ref[:, :16], r16_ref[...], preferred_element_type=_f32,
                    precision=_HIGH)
    o_ref[...] = ms_ref[...] * ex128


def _wrow(ms, exd, r16):
    blk = 4000
    ne = ms.shape[0]
    return pl.pallas_call(
        _wrow_body,
        grid=(ne // blk,),
        in_specs=[
            pl.BlockSpec((blk, D), lambda i: (i, 0)),
            pl.BlockSpec((blk, D), lambda i: (i, 0)),
            pl.BlockSpec((16, D), lambda i: (0, 0)),
        ],
        out_specs=pl.BlockSpec((blk, D), lambda i: (i, 0)),
        out_shape=jax.ShapeDtypeStruct((ne, D), _f32),
    )(ms, exd, r16)


def _mlp_body(aa_ref, ab_ref, w1_ref, b1_ref, g_ref, bb_ref, w2_ref, b2_ref, o_ref):
    a = aa_ref[0] + aa_ref[1] + ab_ref[0] + ab_ref[1]
    h = (
        jnp.dot(a, w1_ref[...], preferred_element_type=_f32, precision=_HIGH)
        + b1_ref[...]
    )
    mean = jnp.mean(h, axis=0, keepdims=True)
    var = jnp.mean(jnp.square(h - mean), axis=0, keepdims=True)
    hn = (h - mean) * jax.lax.rsqrt(var + 1e-5) * g_ref[...] + bb_ref[...]
    hn = jnp.maximum(hn, 0.0)
    o_ref[...] = (
        jnp.dot(hn, w2_ref[...], preferred_element_type=_f32, precision=_HIGH)
        + b2_ref[...]
    )


def _mlp(aggpa, aggpb, W1, b1, gamma, beta, W2, b2):
    return pl.pallas_call(
        _mlp_body,
        out_shape=jax.ShapeDtypeStruct((N, D), _f32),
    )(aggpa, aggpb, W1, b1.reshape(1, D), gamma.reshape(1, D),
      beta.reshape(1, D), W2, b2.reshape(1, D))


def kernel(x, edge_index, Wk, bk, Wm, bm, Wq, bq, W1, b1, gamma, beta, W2, b2):
    src1d = edge_index[0]
    dst1d = edge_index[1]
    inv = 1.0 / math.sqrt(DH)
    wcat = jnp.concatenate([Wk, Wm, Wq * inv], axis=1)
    bcat = jnp.concatenate([bk, bm, bq * inv], axis=0).reshape(1, 3 * D)
    K, M, Q = _proj(x, wcat, bcat)

    d_iota = jnp.arange(D, dtype=jnp.int32)
    s8 = (d_iota[:, None] // DH == jnp.arange(D)[None, :]).astype(_f32)
    r16 = (jnp.arange(16, dtype=jnp.int32)[:, None] == d_iota[None, :] // DH
           ).astype(_f32)

    EH = E // 2
    sA, sB = src1d[:EH], src1d[EH:]
    dA, dB = dst1d[:EH], dst1d[EH:]
    z128 = jnp.zeros((_NPAD, D), _f32)

    qsA, kdA = _sc_gather2(Q, K, sA, dA, EH)
    exdA = _scores(qsA, kdA, s8)
    qsB, kdB = _sc_gather2(Q, K, sB, dB, EH)
    exdB = _scores(qsB, kdB, s8)
    sspA = _sc_scatter_add(exdA, sA, z128, D, EH)
    sspB = _sc_scatter_add(exdB, sB, z128, D, EH)
    msc = _msc(sspA[:, :N], sspB[:, :N], M, r16)
    msA = _sc_gather1(msc, sA, EH)
    wrA = _wrow(msA, exdA, r16)
    msB = _sc_gather1(msc, sB, EH)
    wrB = _wrow(msB, exdB, r16)
    aggpA = _sc_scatter_add(wrA, dA, z128, D, EH)
    aggpB = _sc_scatter_add(wrB, dB, z128, D, EH)
    return _mlp(aggpA[:, :N], aggpB[:, :N], W1, b1, gamma, beta, W2, b2)
